# baseline (speedup 1.0000x reference)
import functools

import jax
import jax.numpy as jnp
from jax import lax
from jax.experimental import pallas as pl
from jax.experimental.pallas import tpu as pltpu
from jax.experimental.pallas import tpu_sc as plsc

_D = 64
_NC = 2
_NS = 16
_NW = _NC * _NS


def _prep_idx(xvm, pidx, par, j, blk):
    for g in range(8):
        v = xvm[j, pl.ds(blk * 128 + g * 16, 16)]
        pidx[pl.ds(g * 16, 16)] = lax.shift_right_logical(v, 1)
        par[pl.ds(g * 16, 16)] = (v & 1) * 64


def _transpose_select(rows, par, t):
    for g in range(8):
        rid = lax.iota(jnp.int32, 16) + g * 16
        pv = par[pl.ds(g * 16, 16)]

        @pl.loop(0, 8)
        def _c8(c8):
            for cu in range(8):
                c = c8 * 8 + cu
                t[c, pl.ds(g * 16, 16)] = plsc.load_gather(
                    rows, [rid, pv + c])


def _body(xT_hbm, wp_hbm, out_hbm, xvm, pidx, par, rows, t0, t1,
          gsem, wsem0, wsem1):
    wid = lax.axis_index("s") * _NC + lax.axis_index("c")
    n_slot, n_batch = xT_hbm.shape
    b_per_w = n_batch // _NW
    nblk = b_per_w // 128
    bw0 = wid * b_per_w

    pltpu.sync_copy(xT_hbm.at[:, pl.ds(bw0, b_per_w)], xvm)

    t = (t0, t1)
    wsem = (wsem0, wsem1)
    nk = n_slot * nblk

    @pl.loop(0, nk // 2)
    def _k(i):
        for sub in range(2):
            k = i * 2 + sub
            j = k // nblk
            blk = k % nblk
            _prep_idx(xvm, pidx, par, j, blk)
            pltpu.async_copy(wp_hbm.at[pidx], rows, gsem).wait()

            @pl.when(k >= 2)
            def _wait_prev():
                kp = k - 2
                pltpu.make_async_copy(
                    t[sub],
                    out_hbm.at[kp // nblk, :,
                               pl.ds(bw0 + (kp % nblk) * 128, 128)],
                    wsem[sub]).wait()

            _transpose_select(rows, par, t[sub])
            pltpu.async_copy(
                t[sub],
                out_hbm.at[j, :, pl.ds(bw0 + blk * 128, 128)],
                wsem[sub])

    for sub in range(2):
        k = nk - 2 + sub
        pltpu.make_async_copy(
            t[sub],
            out_hbm.at[k // nblk, :, pl.ds(bw0 + (k % nblk) * 128, 128)],
            wsem[sub]).wait()


@jax.jit
def _embed(x, weight):
    n_batch, n_slot = x.shape
    n_rows, d = weight.shape
    xT = x.T
    wp = weight.reshape(n_rows // 2, 2 * d)
    run = functools.partial(
        pl.kernel,
        out_type=jax.ShapeDtypeStruct((n_slot, d, n_batch), jnp.float32),
        mesh=plsc.VectorSubcoreMesh(
            core_axis_name="c", subcore_axis_name="s",
            num_cores=_NC, num_subcores=_NS,
        ),
        scratch_types=[
            pltpu.VMEM((n_slot, n_batch // _NW), jnp.int32),
            pltpu.VMEM((128,), jnp.int32),
            pltpu.VMEM((128,), jnp.int32),
            pltpu.VMEM((128, 2 * d), jnp.float32),
            pltpu.VMEM((d, 128), jnp.float32),
            pltpu.VMEM((d, 128), jnp.float32),
            pltpu.SemaphoreType.DMA,
            pltpu.SemaphoreType.DMA,
            pltpu.SemaphoreType.DMA,
        ],
        compiler_params=pltpu.CompilerParams(
            use_tc_tiling_on_sc=True, needs_layout_passes=False),
    )(_body)
    out = run(xT, wp)
    return out.transpose(2, 0, 1)


def kernel(x, weight):
    return _embed(x.astype(jnp.int32), weight)


# speedup vs baseline: 1.0838x; 1.0838x over previous
import functools

import jax
import jax.numpy as jnp
from jax import lax
from jax.experimental import pallas as pl
from jax.experimental.pallas import tpu as pltpu
from jax.experimental.pallas import tpu_sc as plsc

_D = 64
_NC = 2
_NS = 16
_NW = _NC * _NS


def _prep_idx(xvm, pidx, par, j, blk):
    for g in range(8):
        v = xvm[j, pl.ds(blk * 128 + g * 16, 16)]
        pidx[pl.ds(g * 16, 16)] = lax.shift_right_logical(v, 1)
        par[pl.ds(g * 16, 16)] = (v & 1) * 64


def _transpose_select(rows, par, t):
    for g in range(8):
        rid = lax.iota(jnp.int32, 16) + g * 16
        pv = par[pl.ds(g * 16, 16)]
        for c in range(64):
            t[c, pl.ds(g * 16, 16)] = plsc.load_gather(rows, [rid, pv + c])


def _body(xT_hbm, wp_hbm, out_hbm, xvm, pidx0, pidx1, par0, par1,
          rows0, rows1, t0, t1, gsem0, gsem1, wsem0, wsem1):
    wid = lax.axis_index("s") * _NC + lax.axis_index("c")
    n_slot, n_batch = xT_hbm.shape
    b_per_w = n_batch // _NW
    nblk = b_per_w // 128
    bw0 = wid * b_per_w

    pltpu.sync_copy(xT_hbm.at[:, pl.ds(bw0, b_per_w)], xvm)

    pidx = (pidx0, pidx1)
    par = (par0, par1)
    rows = (rows0, rows1)
    t = (t0, t1)
    gsem = (gsem0, gsem1)
    wsem = (wsem0, wsem1)
    nk = n_slot * nblk

    def _out_slice(k):
        return out_hbm.at[k // nblk, :, pl.ds(bw0 + (k % nblk) * 128, 128)]

    _prep_idx(xvm, pidx0, par0, 0, 0)
    pltpu.async_copy(wp_hbm.at[pidx0], rows0, gsem0)

    @pl.loop(0, nk // 2)
    def _k(i):
        for sub in range(2):
            k = i * 2 + sub
            b = sub
            nb = 1 - sub
            pltpu.make_async_copy(
                wp_hbm.at[pidx[b]], rows[b], gsem[b]).wait()

            kn = k + 1

            @pl.when(kn < nk)
            def _fire_next():
                _prep_idx(xvm, pidx[nb], par[nb], kn // nblk, kn % nblk)
                pltpu.async_copy(wp_hbm.at[pidx[nb]], rows[nb], gsem[nb])

            @pl.when(k >= 2)
            def _wait_prev():
                pltpu.make_async_copy(t[b], _out_slice(k - 2),
                                      wsem[b]).wait()

            _transpose_select(rows[b], par[b], t[b])
            pltpu.async_copy(t[b], _out_slice(k), wsem[b])

    for sub in range(2):
        k = nk - 2 + sub
        pltpu.make_async_copy(t[sub], _out_slice(k), wsem[sub]).wait()


@jax.jit
def _embed(x, weight):
    n_batch, n_slot = x.shape
    n_rows, d = weight.shape
    xT = x.T
    wp = weight.reshape(n_rows // 2, 2 * d)
    run = functools.partial(
        pl.kernel,
        out_type=jax.ShapeDtypeStruct((n_slot, d, n_batch), jnp.float32),
        mesh=plsc.VectorSubcoreMesh(
            core_axis_name="c", subcore_axis_name="s",
            num_cores=_NC, num_subcores=_NS,
        ),
        scratch_types=[
            pltpu.VMEM((n_slot, n_batch // _NW), jnp.int32),
            pltpu.VMEM((128,), jnp.int32),
            pltpu.VMEM((128,), jnp.int32),
            pltpu.VMEM((128,), jnp.int32),
            pltpu.VMEM((128,), jnp.int32),
            pltpu.VMEM((128, 2 * d), jnp.float32),
            pltpu.VMEM((128, 2 * d), jnp.float32),
            pltpu.VMEM((d, 128), jnp.float32),
            pltpu.VMEM((d, 128), jnp.float32),
            pltpu.SemaphoreType.DMA,
            pltpu.SemaphoreType.DMA,
            pltpu.SemaphoreType.DMA,
            pltpu.SemaphoreType.DMA,
        ],
        compiler_params=pltpu.CompilerParams(
            use_tc_tiling_on_sc=True, needs_layout_passes=False),
    )(_body)
    out = run(xT, wp)
    return out.transpose(2, 0, 1)


def kernel(x, weight):
    return _embed(x.astype(jnp.int32), weight)


# speedup vs baseline: 1.4167x; 1.3072x over previous
import functools

import jax
import jax.numpy as jnp
from jax import lax
from jax.experimental import pallas as pl
from jax.experimental.pallas import tpu as pltpu
from jax.experimental.pallas import tpu_sc as plsc

_D = 64
_NC = 2
_NS = 16
_NW = _NC * _NS


def _prep_idx(xvm, pidx, par, j, blk):
    for g in range(8):
        v = xvm[j, pl.ds(blk * 128 + g * 16, 16)]
        pidx[pl.ds(g * 16, 16)] = lax.shift_right_logical(v, 1)
        par[pl.ds(g * 16, 16)] = (v & 1) * 64


def _transpose_select(rows, par, t):
    for g in range(8):
        rid = lax.iota(jnp.int32, 16) + g * 16
        pv = par[pl.ds(g * 16, 16)]
        for c8 in range(8):
            vals = [plsc.load_gather(rows, [rid, pv + (c8 * 8 + cu)])
                    for cu in range(8)]
            for cu in range(8):
                t[c8 * 8 + cu, pl.ds(g * 16, 16)] = vals[cu]


def _body(xT_hbm, wp_hbm, out_hbm, xvm, pidx0, pidx1, par0, par1,
          rows0, rows1, t0, t1, gsem0, gsem1, wsem0, wsem1):
    wid = lax.axis_index("s") * _NC + lax.axis_index("c")
    n_slot, n_batch = xT_hbm.shape
    b_per_w = n_batch // _NW
    nblk = b_per_w // 128
    bw0 = wid * b_per_w

    pltpu.sync_copy(xT_hbm.at[:, pl.ds(bw0, b_per_w)], xvm)

    pidx = (pidx0, pidx1)
    par = (par0, par1)
    rows = (rows0, rows1)
    t = (t0, t1)
    gsem = (gsem0, gsem1)
    wsem = (wsem0, wsem1)
    nk = n_slot * nblk

    def _out_slice(k):
        return out_hbm.at[k // nblk, :, pl.ds(bw0 + (k % nblk) * 128, 128)]

    _prep_idx(xvm, pidx0, par0, 0, 0)
    pltpu.async_copy(wp_hbm.at[pidx0], rows0, gsem0)

    @pl.loop(0, nk // 2)
    def _k(i):
        for sub in range(2):
            k = i * 2 + sub
            b = sub
            nb = 1 - sub
            pltpu.make_async_copy(
                wp_hbm.at[pidx[b]], rows[b], gsem[b]).wait()

            kn = k + 1

            @pl.when(kn < nk)
            def _fire_next():
                _prep_idx(xvm, pidx[nb], par[nb], kn // nblk, kn % nblk)
                pltpu.async_copy(wp_hbm.at[pidx[nb]], rows[nb], gsem[nb])

            @pl.when(k >= 2)
            def _wait_prev():
                pltpu.make_async_copy(t[b], _out_slice(k - 2),
                                      wsem[b]).wait()

            _transpose_select(rows[b], par[b], t[b])
            pltpu.async_copy(t[b], _out_slice(k), wsem[b])

    for sub in range(2):
        k = nk - 2 + sub
        pltpu.make_async_copy(t[sub], _out_slice(k), wsem[sub]).wait()


@jax.jit
def _embed(x, weight):
    n_batch, n_slot = x.shape
    n_rows, d = weight.shape
    xT = x.T
    wp = weight.reshape(n_rows // 2, 2 * d)
    run = functools.partial(
        pl.kernel,
        out_type=jax.ShapeDtypeStruct((n_slot, d, n_batch), jnp.float32),
        mesh=plsc.VectorSubcoreMesh(
            core_axis_name="c", subcore_axis_name="s",
            num_cores=_NC, num_subcores=_NS,
        ),
        scratch_types=[
            pltpu.VMEM((n_slot, n_batch // _NW), jnp.int32),
            pltpu.VMEM((128,), jnp.int32),
            pltpu.VMEM((128,), jnp.int32),
            pltpu.VMEM((128,), jnp.int32),
            pltpu.VMEM((128,), jnp.int32),
            pltpu.VMEM((128, 2 * d), jnp.float32),
            pltpu.VMEM((128, 2 * d), jnp.float32),
            pltpu.VMEM((d, 128), jnp.float32),
            pltpu.VMEM((d, 128), jnp.float32),
            pltpu.SemaphoreType.DMA,
            pltpu.SemaphoreType.DMA,
            pltpu.SemaphoreType.DMA,
            pltpu.SemaphoreType.DMA,
        ],
        compiler_params=pltpu.CompilerParams(
            use_tc_tiling_on_sc=True, needs_layout_passes=False),
    )(_body)
    out = run(xT, wp)
    return out.transpose(2, 0, 1)


def kernel(x, weight):
    return _embed(x.astype(jnp.int32), weight)
